# baseline (speedup 1.0000x reference)
import functools

import jax
import jax.numpy as jnp
from jax import lax
from jax.experimental import pallas as pl
from jax.experimental.pallas import tpu as pltpu
from jax.experimental.pallas import tpu_sc as plsc

NC = 2
NS = 16
L = 16
NW = NC * NS

EDGE_CHUNK = 40


def _proj_body(x_ref, w_ref, b_ref, h_ref, q_ref, kv_ref):
    d = x_ref.shape[1]
    p = jnp.dot(x_ref[...], w_ref[...], preferred_element_type=jnp.float32)
    p = p + b_ref[...]
    h_ref[...] = p[:, 0 * d:1 * d]
    nl2e = jnp.float32(-1.0)
    q_ref[...] = p[:, 1 * d:2 * d] * nl2e
    kb = lax.bitcast_convert_type(p[:, 2 * d:3 * d] * nl2e, jnp.int32)
    vb = lax.bitcast_convert_type(p[:, 3 * d:4 * d], jnp.int32)
    kr = (kb + 0x8000) & jnp.int32(-65536)
    vr = lax.shift_right_logical(vb + 0x8000, 16)
    kv_ref[...] = lax.bitcast_convert_type(kr | vr, jnp.float32)


def _add_body(h_ref, p0_ref, p1_ref, o_ref):
    o_ref[...] = h_ref[...] + p0_ref[0] + p1_ref[0]


def _make_edge_kernel(n_nodes, n_edges, d):
    epw = n_edges // NW
    c = EDGE_CHUNK
    nchunk = epw // c
    assert nchunk % 4 == 2 and nchunk >= 6
    nquads = (nchunk - 2) // 4
    rpt = (n_nodes // NS) // 8 * 8
    tail = n_nodes - rpt * NS

    mesh = plsc.VectorSubcoreMesh(core_axis_name="c", subcore_axis_name="s")

    @functools.partial(
        pl.kernel,
        out_type=jax.ShapeDtypeStruct((NC, n_nodes, d), jnp.float32),
        mesh=mesh,
        compiler_params=pltpu.CompilerParams(needs_layout_passes=False),
        scratch_types=[
            pltpu.VMEM((2, c), jnp.int32),
            pltpu.VMEM((2, c), jnp.int32),
            pltpu.VMEM((2, c), jnp.int32),
            pltpu.VMEM((2, c), jnp.int32),
            pltpu.VMEM((c, d), jnp.float32),
            pltpu.VMEM((c, d), jnp.float32),
            pltpu.VMEM((c, d), jnp.float32),
            pltpu.VMEM((c, d), jnp.float32),
            pltpu.VMEM_SHARED((n_nodes, d), jnp.float32),
            pltpu.SemaphoreType.DMA,
            pltpu.SemaphoreType.DMA,
            pltpu.SemaphoreType.DMA,
            pltpu.SemaphoreType.DMA,
            pltpu.SemaphoreType.DMA,
            pltpu.SemaphoreType.DMA,
        ],
    )
    def edge_kernel(q_hbm, kv_hbm, sr_hbm, zero_hbm, out_hbm,
                    s0, s1, s2, s3, qa, kva, qb, kvb, acc,
                    ss0, ss1, ss2, ss3, sem_a, sem_b):
        cid = lax.axis_index("c")
        sid = lax.axis_index("s")
        wid = sid * NC + cid
        srs = ((s0, ss0), (s1, ss1), (s2, ss2), (s3, ss3))
        sets = ((qa, kva, sem_a), (qb, kvb, sem_b))

        pltpu.sync_copy(zero_hbm.at[pl.ds(sid * rpt, rpt)],
                        acc.at[pl.ds(sid * rpt, rpt)])
        if tail:
            @pl.when(sid == NS - 1)
            def _():
                pltpu.sync_copy(zero_hbm.at[pl.ds(rpt * NS, tail)],
                                acc.at[pl.ds(rpt * NS, tail)])
        plsc.subcore_barrier()

        def fire_sr(g, slot):
            sr, sem = srs[slot]
            pltpu.async_copy(sr_hbm.at[wid, g], sr, sem)

        def wait_sr(slot):
            sr, sem = srs[slot]
            pltpu.make_async_copy(sr_hbm.at[wid, 0], sr, sem).wait()

        def fire_gather(slot, st):
            sr, _ = srs[slot]
            qx, kvx, sem = sets[st]
            pltpu.async_copy(q_hbm.at[sr.at[1]], qx, sem)
            pltpu.async_copy(kv_hbm.at[sr.at[0]], kvx, sem)

        def process(slot, st):
            sr, _ = srs[slot]
            qx, kvx, sem = sets[st]
            dummy = q_hbm.at[pl.ds(0, c)]
            pltpu.make_async_copy(dummy, qx, sem).wait()
            pltpu.make_async_copy(dummy, kvx, sem).wait()

            @plsc.parallel_loop(0, c, step=1, unroll=2)
            def edge_body(i):
                for j in range(d // L):
                    sl = pl.ds(j * L, L)
                    ab = plsc.bitcast(kvx[i, sl], jnp.bfloat16)
                    vf, kf = plsc.unpack(
                        ab, format=plsc.PackFormat.INTERLEAVED,
                        preferred_element_type=jnp.float32)
                    eta = 1.0 / (1.0 + jnp.exp(qx[i, sl] + kf))
                    kvx[i, sl] = eta * vf
            pltpu.sync_copy(kvx, acc.at[sr.at[1]], add=True)

        fire_sr(0, 0)
        fire_sr(1, 1)
        fire_sr(2, 2)
        fire_sr(3, 3)
        wait_sr(0)
        fire_gather(0, 0)

        def quad_body(i, carry):
            c0 = 4 * i
            wait_sr(1)
            fire_gather(1, 1)
            process(0, 0)
            fire_sr(c0 + 4, 0)
            wait_sr(2)
            fire_gather(2, 0)
            process(1, 1)
            fire_sr(c0 + 5, 1)
            wait_sr(3)
            fire_gather(3, 1)
            process(2, 0)

            @pl.when(c0 + 6 < nchunk)
            def _():
                fire_sr(c0 + 6, 2)

            wait_sr(0)
            fire_gather(0, 0)
            process(3, 1)

            @pl.when(c0 + 7 < nchunk)
            def _():
                fire_sr(c0 + 7, 3)

            return carry

        lax.fori_loop(0, nquads, quad_body, 0)
        wait_sr(1)
        fire_gather(1, 1)
        process(0, 0)
        process(1, 1)

        plsc.subcore_barrier()
        pltpu.sync_copy(acc.at[pl.ds(sid * rpt, rpt)],
                        out_hbm.at[cid, pl.ds(sid * rpt, rpt)])
        if tail:
            @pl.when(sid == NS - 1)
            def _():
                pltpu.sync_copy(acc.at[pl.ds(rpt * NS, tail)],
                                out_hbm.at[cid, pl.ds(rpt * NS, tail)])

    return edge_kernel


def kernel(node_features, senders, receivers, W_kernel, W_bias):
    n, d = node_features.shape
    e = senders.shape[0]
    senders = senders.astype(jnp.int32)
    receivers = receivers.astype(jnp.int32)

    blk = 1000
    grid = n // blk
    h, q, kv = pl.pallas_call(
        _proj_body,
        grid=(grid,),
        in_specs=[
            pl.BlockSpec((blk, d), lambda i: (i, 0)),
            pl.BlockSpec((d, 4 * d), lambda i: (0, 0)),
            pl.BlockSpec((1, 4 * d), lambda i: (0, 0)),
        ],
        out_specs=[pl.BlockSpec((blk, d), lambda i: (i, 0)) for _ in range(3)],
        out_shape=[jax.ShapeDtypeStruct((n, d), jnp.float32) for _ in range(3)],
    )(node_features, W_kernel, W_bias.reshape(1, 4 * d))

    zeros = jnp.zeros((n, d), jnp.float32)
    epw = e // NW
    nchunk = epw // EDGE_CHUNK
    sr = jnp.stack(
        (senders.reshape(NW, nchunk, EDGE_CHUNK),
         receivers.reshape(NW, nchunk, EDGE_CHUNK)), axis=2)
    part = _make_edge_kernel(n, e, d)(q, kv, sr, zeros)

    out = pl.pallas_call(
        _add_body,
        grid=(grid,),
        in_specs=[
            pl.BlockSpec((blk, d), lambda i: (i, 0)),
            pl.BlockSpec((1, blk, d), lambda i: (0, i, 0)),
            pl.BlockSpec((1, blk, d), lambda i: (1, i, 0)),
        ],
        out_specs=pl.BlockSpec((blk, d), lambda i: (i, 0)),
        out_shape=jax.ShapeDtypeStruct((n, d), jnp.float32),
    )(h, part, part)
    return out


# speedup vs baseline: 1.1739x; 1.1739x over previous
import functools

import jax
import jax.numpy as jnp
from jax import lax
from jax.experimental import pallas as pl
from jax.experimental.pallas import tpu as pltpu
from jax.experimental.pallas import tpu_sc as plsc

NC = 2
NS = 16
L = 16
NW = NC * NS

EDGE_CHUNK = 40


def _proj_body(x_ref, w_ref, b_ref, h_ref, q_ref, k_ref, v_ref):
    d = x_ref.shape[1]
    p = jnp.dot(x_ref[...], w_ref[...], preferred_element_type=jnp.float32)
    p = p + b_ref[...]
    h_ref[...] = p[:, 0 * d:1 * d]
    q_ref[...] = -p[:, 1 * d:2 * d]
    k_ref[...] = -p[:, 2 * d:3 * d]
    v_ref[...] = p[:, 3 * d:4 * d]


def _add_body(h_ref, p0_ref, p1_ref, o_ref):
    o_ref[...] = h_ref[...] + p0_ref[0] + p1_ref[0]


def _make_edge_kernel(n_nodes, n_edges, d):
    epw = n_edges // NW
    c = EDGE_CHUNK
    nchunk = epw // c
    assert nchunk % 4 == 2 and nchunk >= 6
    nquads = (nchunk - 2) // 4
    rpt = (n_nodes // NS) // 8 * 8
    tail = n_nodes - rpt * NS

    mesh = plsc.VectorSubcoreMesh(core_axis_name="c", subcore_axis_name="s")

    @functools.partial(
        pl.kernel,
        out_type=jax.ShapeDtypeStruct((NC, n_nodes, d), jnp.float32),
        mesh=mesh,
        compiler_params=pltpu.CompilerParams(needs_layout_passes=False),
        scratch_types=[
            pltpu.VMEM((2, c), jnp.int32),
            pltpu.VMEM((2, c), jnp.int32),
            pltpu.VMEM((2, c), jnp.int32),
            pltpu.VMEM((2, c), jnp.int32),
            pltpu.VMEM((c, d), jnp.float32),
            pltpu.VMEM((c, d), jnp.float32),
            pltpu.VMEM((c, d), jnp.float32),
            pltpu.VMEM((c, d), jnp.float32),
            pltpu.VMEM((c, d), jnp.float32),
            pltpu.VMEM((c, d), jnp.float32),
            pltpu.VMEM_SHARED((n_nodes, d), jnp.float32),
            pltpu.SemaphoreType.DMA,
            pltpu.SemaphoreType.DMA,
            pltpu.SemaphoreType.DMA,
            pltpu.SemaphoreType.DMA,
            pltpu.SemaphoreType.DMA,
            pltpu.SemaphoreType.DMA,
        ],
    )
    def edge_kernel(q_hbm, k_hbm, v_hbm, sr_hbm, zero_hbm, out_hbm,
                    s0, s1, s2, s3, qa, ka, va, qb, kb, vb, acc,
                    ss0, ss1, ss2, ss3, sem_a, sem_b):
        cid = lax.axis_index("c")
        sid = lax.axis_index("s")
        wid = sid * NC + cid
        srs = ((s0, ss0), (s1, ss1), (s2, ss2), (s3, ss3))
        sets = ((qa, ka, va, sem_a), (qb, kb, vb, sem_b))

        pltpu.sync_copy(zero_hbm.at[pl.ds(sid * rpt, rpt)],
                        acc.at[pl.ds(sid * rpt, rpt)])
        if tail:
            @pl.when(sid == NS - 1)
            def _():
                pltpu.sync_copy(zero_hbm.at[pl.ds(rpt * NS, tail)],
                                acc.at[pl.ds(rpt * NS, tail)])
        plsc.subcore_barrier()

        def fire_sr(g, slot):
            sr, sem = srs[slot]
            pltpu.async_copy(sr_hbm.at[wid, g], sr, sem)

        def wait_sr(slot):
            sr, sem = srs[slot]
            pltpu.make_async_copy(sr_hbm.at[wid, 0], sr, sem).wait()

        def fire_gather(slot, st):
            sr, _ = srs[slot]
            qx, kx, vx, sem = sets[st]
            pltpu.async_copy(q_hbm.at[sr.at[1]], qx, sem)
            pltpu.async_copy(k_hbm.at[sr.at[0]], kx, sem)
            pltpu.async_copy(v_hbm.at[sr.at[0]], vx, sem)

        def process(slot, st):
            sr, _ = srs[slot]
            qx, kx, vx, sem = sets[st]
            dummy = q_hbm.at[pl.ds(0, c)]
            pltpu.make_async_copy(dummy, qx, sem).wait()
            pltpu.make_async_copy(dummy, kx, sem).wait()
            pltpu.make_async_copy(dummy, vx, sem).wait()

            def edge_body(i2, carry2):
                for u in range(2):
                    i = i2 * 2 + u
                    for j in range(d // L):
                        sl = pl.ds(j * L, L)
                        eta = 1.0 / (1.0 + jnp.exp(qx[i, sl] + kx[i, sl]))
                        vx[i, sl] = eta * vx[i, sl]
                return carry2

            lax.fori_loop(0, c // 2, edge_body, 0)
            pltpu.sync_copy(vx, acc.at[sr.at[1]], add=True)

        fire_sr(0, 0)
        fire_sr(1, 1)
        fire_sr(2, 2)
        fire_sr(3, 3)
        wait_sr(0)
        fire_gather(0, 0)

        def quad_body(i, carry):
            c0 = 4 * i
            wait_sr(1)
            fire_gather(1, 1)
            process(0, 0)
            fire_sr(c0 + 4, 0)
            wait_sr(2)
            fire_gather(2, 0)
            process(1, 1)
            fire_sr(c0 + 5, 1)
            wait_sr(3)
            fire_gather(3, 1)
            process(2, 0)

            @pl.when(c0 + 6 < nchunk)
            def _():
                fire_sr(c0 + 6, 2)

            wait_sr(0)
            fire_gather(0, 0)
            process(3, 1)

            @pl.when(c0 + 7 < nchunk)
            def _():
                fire_sr(c0 + 7, 3)

            return carry

        lax.fori_loop(0, nquads, quad_body, 0)
        wait_sr(1)
        fire_gather(1, 1)
        process(0, 0)
        process(1, 1)

        plsc.subcore_barrier()
        pltpu.sync_copy(acc.at[pl.ds(sid * rpt, rpt)],
                        out_hbm.at[cid, pl.ds(sid * rpt, rpt)])
        if tail:
            @pl.when(sid == NS - 1)
            def _():
                pltpu.sync_copy(acc.at[pl.ds(rpt * NS, tail)],
                                out_hbm.at[cid, pl.ds(rpt * NS, tail)])

    return edge_kernel


def kernel(node_features, senders, receivers, W_kernel, W_bias):
    n, d = node_features.shape
    e = senders.shape[0]
    senders = senders.astype(jnp.int32)
    receivers = receivers.astype(jnp.int32)

    blk = 1000
    grid = n // blk
    h, q, k, v = pl.pallas_call(
        _proj_body,
        grid=(grid,),
        in_specs=[
            pl.BlockSpec((blk, d), lambda i: (i, 0)),
            pl.BlockSpec((d, 4 * d), lambda i: (0, 0)),
            pl.BlockSpec((1, 4 * d), lambda i: (0, 0)),
        ],
        out_specs=[pl.BlockSpec((blk, d), lambda i: (i, 0)) for _ in range(4)],
        out_shape=[jax.ShapeDtypeStruct((n, d), jnp.float32) for _ in range(4)],
    )(node_features, W_kernel, W_bias.reshape(1, 4 * d))

    zeros = jnp.zeros((n, d), jnp.float32)
    epw = e // NW
    nchunk = epw // EDGE_CHUNK
    sr = jnp.stack(
        (senders.reshape(NW, nchunk, EDGE_CHUNK),
         receivers.reshape(NW, nchunk, EDGE_CHUNK)), axis=2)
    part = _make_edge_kernel(n, e, d)(q, k, v, sr, zeros)

    out = pl.pallas_call(
        _add_body,
        grid=(grid,),
        in_specs=[
            pl.BlockSpec((blk, d), lambda i: (i, 0)),
            pl.BlockSpec((1, blk, d), lambda i: (0, i, 0)),
            pl.BlockSpec((1, blk, d), lambda i: (1, i, 0)),
        ],
        out_specs=pl.BlockSpec((blk, d), lambda i: (i, 0)),
        out_shape=jax.ShapeDtypeStruct((n, d), jnp.float32),
    )(h, part, part)
    return out


# speedup vs baseline: 1.2971x; 1.1050x over previous
import functools

import jax
import jax.numpy as jnp
from jax import lax
from jax.experimental import pallas as pl
from jax.experimental.pallas import tpu as pltpu
from jax.experimental.pallas import tpu_sc as plsc

NC = 2
NS = 16
L = 16
NW = NC * NS

EDGE_CHUNK = 40


def _proj_body(x_ref, w_ref, b_ref, h_ref, q_ref, kv_ref):
    d = x_ref.shape[1]
    p = jnp.dot(x_ref[...], w_ref[...], preferred_element_type=jnp.float32)
    p = p + b_ref[...]
    h_ref[...] = p[:, 0 * d:1 * d]
    q_ref[...] = -p[:, 1 * d:2 * d]
    kb = lax.bitcast_convert_type(-p[:, 2 * d:3 * d], jnp.int32)
    vb = lax.bitcast_convert_type(p[:, 3 * d:4 * d], jnp.int32)
    kr = (kb + 0x8000) & jnp.int32(-65536)
    vr = lax.shift_right_logical(vb + 0x8000, 16)
    kv_ref[...] = lax.bitcast_convert_type(kr | vr, jnp.float32)


def _add_body(h_ref, p0_ref, p1_ref, o_ref):
    o_ref[...] = h_ref[...] + p0_ref[0] + p1_ref[0]


def _make_edge_kernel(n_nodes, n_edges, d):
    epw = n_edges // NW
    c = EDGE_CHUNK
    nchunk = epw // c
    assert nchunk % 4 == 2 and nchunk >= 6
    nquads = (nchunk - 2) // 4
    rpt = (n_nodes // NS) // 8 * 8
    tail = n_nodes - rpt * NS

    mesh = plsc.VectorSubcoreMesh(core_axis_name="c", subcore_axis_name="s")

    @functools.partial(
        pl.kernel,
        out_type=jax.ShapeDtypeStruct((NC, n_nodes, d), jnp.float32),
        mesh=mesh,
        compiler_params=pltpu.CompilerParams(needs_layout_passes=False),
        scratch_types=[
            pltpu.VMEM((2, c), jnp.int32),
            pltpu.VMEM((2, c), jnp.int32),
            pltpu.VMEM((2, c), jnp.int32),
            pltpu.VMEM((2, c), jnp.int32),
            pltpu.VMEM((c, d), jnp.float32),
            pltpu.VMEM((c, d), jnp.float32),
            pltpu.VMEM((c, d), jnp.float32),
            pltpu.VMEM((c, d), jnp.float32),
            pltpu.VMEM_SHARED((n_nodes, d), jnp.float32),
            pltpu.SemaphoreType.DMA,
            pltpu.SemaphoreType.DMA,
            pltpu.SemaphoreType.DMA,
            pltpu.SemaphoreType.DMA,
            pltpu.SemaphoreType.DMA,
            pltpu.SemaphoreType.DMA,
        ],
    )
    def edge_kernel(q_hbm, kv_hbm, sr_hbm, zero_hbm, out_hbm,
                    s0, s1, s2, s3, qa, kva, qb, kvb, acc,
                    ss0, ss1, ss2, ss3, sem_a, sem_b):
        cid = lax.axis_index("c")
        sid = lax.axis_index("s")
        wid = sid * NC + cid
        srs = ((s0, ss0), (s1, ss1), (s2, ss2), (s3, ss3))
        sets = ((qa, kva, sem_a), (qb, kvb, sem_b))

        pltpu.sync_copy(zero_hbm.at[pl.ds(sid * rpt, rpt)],
                        acc.at[pl.ds(sid * rpt, rpt)])
        if tail:
            @pl.when(sid == NS - 1)
            def _():
                pltpu.sync_copy(zero_hbm.at[pl.ds(rpt * NS, tail)],
                                acc.at[pl.ds(rpt * NS, tail)])
        plsc.subcore_barrier()

        def fire_sr(g, slot):
            sr, sem = srs[slot]
            pltpu.async_copy(sr_hbm.at[wid, g], sr, sem)

        def wait_sr(slot):
            sr, sem = srs[slot]
            pltpu.make_async_copy(sr_hbm.at[wid, 0], sr, sem).wait()

        def fire_gather(slot, st):
            sr, _ = srs[slot]
            qx, kvx, sem = sets[st]
            pltpu.async_copy(q_hbm.at[sr.at[1]], qx, sem)
            pltpu.async_copy(kv_hbm.at[sr.at[0]], kvx, sem)

        def process(slot, st):
            sr, _ = srs[slot]
            qx, kvx, sem = sets[st]
            dummy = q_hbm.at[pl.ds(0, c)]
            pltpu.make_async_copy(dummy, qx, sem).wait()
            pltpu.make_async_copy(dummy, kvx, sem).wait()

            def edge_body(i2, carry2):
                for u in range(2):
                    i = i2 * 2 + u
                    for j in range(d // L):
                        sl = pl.ds(j * L, L)
                        w = plsc.bitcast(kvx[i, sl], jnp.int32)
                        kf = plsc.bitcast(w, jnp.float32)
                        vf = plsc.bitcast(
                            lax.shift_left(w, jnp.full((L,), 16, jnp.int32)),
                            jnp.float32)
                        eta = 1.0 / (1.0 + jnp.exp(qx[i, sl] + kf))
                        kvx[i, sl] = eta * vf
                return carry2

            lax.fori_loop(0, c // 2, edge_body, 0)
            pltpu.sync_copy(kvx, acc.at[sr.at[1]], add=True)

        fire_sr(0, 0)
        fire_sr(1, 1)
        fire_sr(2, 2)
        fire_sr(3, 3)
        wait_sr(0)
        fire_gather(0, 0)

        def quad_body(i, carry):
            c0 = 4 * i
            wait_sr(1)
            fire_gather(1, 1)
            process(0, 0)
            fire_sr(c0 + 4, 0)
            wait_sr(2)
            fire_gather(2, 0)
            process(1, 1)
            fire_sr(c0 + 5, 1)
            wait_sr(3)
            fire_gather(3, 1)
            process(2, 0)

            @pl.when(c0 + 6 < nchunk)
            def _():
                fire_sr(c0 + 6, 2)

            wait_sr(0)
            fire_gather(0, 0)
            process(3, 1)

            @pl.when(c0 + 7 < nchunk)
            def _():
                fire_sr(c0 + 7, 3)

            return carry

        lax.fori_loop(0, nquads, quad_body, 0)
        wait_sr(1)
        fire_gather(1, 1)
        process(0, 0)
        process(1, 1)

        plsc.subcore_barrier()
        pltpu.sync_copy(acc.at[pl.ds(sid * rpt, rpt)],
                        out_hbm.at[cid, pl.ds(sid * rpt, rpt)])
        if tail:
            @pl.when(sid == NS - 1)
            def _():
                pltpu.sync_copy(acc.at[pl.ds(rpt * NS, tail)],
                                out_hbm.at[cid, pl.ds(rpt * NS, tail)])

    return edge_kernel


def kernel(node_features, senders, receivers, W_kernel, W_bias):
    n, d = node_features.shape
    e = senders.shape[0]
    senders = senders.astype(jnp.int32)
    receivers = receivers.astype(jnp.int32)

    blk = 1000
    grid = n // blk
    h, q, kv = pl.pallas_call(
        _proj_body,
        grid=(grid,),
        in_specs=[
            pl.BlockSpec((blk, d), lambda i: (i, 0)),
            pl.BlockSpec((d, 4 * d), lambda i: (0, 0)),
            pl.BlockSpec((1, 4 * d), lambda i: (0, 0)),
        ],
        out_specs=[pl.BlockSpec((blk, d), lambda i: (i, 0)) for _ in range(3)],
        out_shape=[jax.ShapeDtypeStruct((n, d), jnp.float32) for _ in range(3)],
    )(node_features, W_kernel, W_bias.reshape(1, 4 * d))

    zeros = jnp.zeros((n, d), jnp.float32)
    epw = e // NW
    nchunk = epw // EDGE_CHUNK
    sr = jnp.stack(
        (senders.reshape(NW, nchunk, EDGE_CHUNK),
         receivers.reshape(NW, nchunk, EDGE_CHUNK)), axis=2)
    part = _make_edge_kernel(n, e, d)(q, kv, sr, zeros)

    out = pl.pallas_call(
        _add_body,
        grid=(grid,),
        in_specs=[
            pl.BlockSpec((blk, d), lambda i: (i, 0)),
            pl.BlockSpec((1, blk, d), lambda i: (0, i, 0)),
            pl.BlockSpec((1, blk, d), lambda i: (1, i, 0)),
        ],
        out_specs=pl.BlockSpec((blk, d), lambda i: (i, 0)),
        out_shape=jax.ShapeDtypeStruct((n, d), jnp.float32),
    )(h, part, part)
    return out


# speedup vs baseline: 1.4163x; 1.0919x over previous
import functools

import jax
import jax.numpy as jnp
from jax import lax
from jax.experimental import pallas as pl
from jax.experimental.pallas import tpu as pltpu
from jax.experimental.pallas import tpu_sc as plsc

NC = 2
NS = 16
L = 16
NW = NC * NS

EDGE_CHUNK = 40


def _proj_body(x_ref, w_ref, b_ref, h_ref, q_ref, kv_ref):
    d = x_ref.shape[1]
    p = jnp.dot(x_ref[...], w_ref[...], preferred_element_type=jnp.float32)
    p = p + b_ref[...]
    h_ref[...] = p[:, 0 * d:1 * d]
    q_ref[...] = -p[:, 1 * d:2 * d]
    kb = lax.bitcast_convert_type(-p[:, 2 * d:3 * d], jnp.int32)
    vb = lax.bitcast_convert_type(p[:, 3 * d:4 * d], jnp.int32)
    kr = (kb + 0x8000) & jnp.int32(-65536)
    vr = lax.shift_right_logical(vb + 0x8000, 16)
    kv_ref[...] = lax.bitcast_convert_type(kr | vr, jnp.float32)


def _add_body(h_ref, p0_ref, p1_ref, o_ref):
    o_ref[...] = h_ref[...] + p0_ref[0] + p1_ref[0]


def _make_edge_kernel(n_nodes, n_edges, d):
    epw = n_edges // NW
    c = EDGE_CHUNK
    nchunk = epw // c
    assert nchunk % 6 == 4 and nchunk >= 16
    rpt = (n_nodes // NS) // 8 * 8
    tail = n_nodes - rpt * NS

    mesh = plsc.VectorSubcoreMesh(core_axis_name="c", subcore_axis_name="s")

    @functools.partial(
        pl.kernel,
        out_type=jax.ShapeDtypeStruct((NC, n_nodes, d), jnp.float32),
        mesh=mesh,
        compiler_params=pltpu.CompilerParams(needs_layout_passes=False),
        scratch_types=(
            [pltpu.VMEM((2, c), jnp.int32) for _ in range(6)]
            + [pltpu.VMEM((c, d), jnp.float32) for _ in range(6)]
            + [pltpu.VMEM_SHARED((n_nodes, d), jnp.float32)]
            + [pltpu.SemaphoreType.DMA for _ in range(12)]
        ),
    )
    def edge_kernel(q_hbm, kv_hbm, sr_hbm, zero_hbm, out_hbm,
                    s0, s1, s2, s3, s4, s5, qa, kva, qb, kvb, qc, kvc, acc,
                    ss0, ss1, ss2, ss3, ss4, ss5,
                    sga, sgb, sgc, sca, scb, scc):
        cid = lax.axis_index("c")
        sid = lax.axis_index("s")
        wid = sid * NC + cid
        srs = ((s0, ss0), (s1, ss1), (s2, ss2), (s3, ss3), (s4, ss4),
               (s5, ss5))
        sets = ((qa, kva, sga, sca), (qb, kvb, sgb, scb), (qc, kvc, sgc, scc))

        pltpu.sync_copy(zero_hbm.at[pl.ds(sid * rpt, rpt)],
                        acc.at[pl.ds(sid * rpt, rpt)])
        if tail:
            @pl.when(sid == NS - 1)
            def _():
                pltpu.sync_copy(zero_hbm.at[pl.ds(rpt * NS, tail)],
                                acc.at[pl.ds(rpt * NS, tail)])
        plsc.subcore_barrier()

        def fire_sr(g, slot):
            sr, sem = srs[slot]
            pltpu.async_copy(sr_hbm.at[wid, g], sr, sem)

        def wait_sr(slot):
            sr, sem = srs[slot]
            pltpu.make_async_copy(sr_hbm.at[wid, 0], sr, sem).wait()

        dummy = q_hbm.at[pl.ds(0, c)]

        def fire_gather(slot, st):
            sr, _ = srs[slot]
            qx, kvx, sem, _ = sets[st]
            pltpu.async_copy(q_hbm.at[sr.at[1]], qx, sem)
            pltpu.async_copy(kv_hbm.at[sr.at[0]], kvx, sem)

        def wait_scat(st):
            qx, kvx, _, sem = sets[st]
            pltpu.make_async_copy(dummy, kvx, sem).wait()

        def process(slot, st):
            sr, _ = srs[slot]
            qx, kvx, sem, scsem = sets[st]
            pltpu.make_async_copy(dummy, qx, sem).wait()
            pltpu.make_async_copy(dummy, kvx, sem).wait()

            def edge_body(i2, carry2):
                for u in range(2):
                    i = i2 * 2 + u
                    for j in range(d // L):
                        sl = pl.ds(j * L, L)
                        w = plsc.bitcast(kvx[i, sl], jnp.int32)
                        kf = plsc.bitcast(w, jnp.float32)
                        vf = plsc.bitcast(
                            lax.shift_left(w, jnp.full((L,), 16, jnp.int32)),
                            jnp.float32)
                        eta = 1.0 / (1.0 + jnp.exp(qx[i, sl] + kf))
                        kvx[i, sl] = eta * vf
                return carry2

            lax.fori_loop(0, c // 2, edge_body, 0)
            pltpu.async_copy(kvx, acc.at[sr.at[1]], scsem, add=True)

        for t in range(6):
            fire_sr(t, t)
        wait_sr(0)
        fire_gather(0, 0)

        for u in range(6):
            wait_sr((u + 1) % 6)
            if u >= 2:
                wait_scat((u + 1) % 3)
            fire_gather((u + 1) % 6, (u + 1) % 3)
            if u >= 2:
                fire_sr(u + 4, (u + 4) % 6)
            process(u % 6, u % 3)

        def hexad_body(i, carry):
            c0 = 6 * i
            for u in range(6):
                wait_sr((u + 1) % 6)
                wait_scat((u + 1) % 3)
                fire_gather((u + 1) % 6, (u + 1) % 3)
                fire_sr(c0 + u + 4, (u + 4) % 6)
                process(u % 6, u % 3)
            return carry

        lax.fori_loop(1, (nchunk - 10) // 6 + 1, hexad_body, 0)

        for u in range(3):
            wait_sr((u + 1) % 6)
            wait_scat((u + 1) % 3)
            fire_gather((u + 1) % 6, (u + 1) % 3)
            process(u % 6, u % 3)
        process(3, 0)
        wait_scat(1)
        wait_scat(2)
        wait_scat(0)

        plsc.subcore_barrier()
        pltpu.sync_copy(acc.at[pl.ds(sid * rpt, rpt)],
                        out_hbm.at[cid, pl.ds(sid * rpt, rpt)])
        if tail:
            @pl.when(sid == NS - 1)
            def _():
                pltpu.sync_copy(acc.at[pl.ds(rpt * NS, tail)],
                                out_hbm.at[cid, pl.ds(rpt * NS, tail)])

    return edge_kernel


def kernel(node_features, senders, receivers, W_kernel, W_bias):
    n, d = node_features.shape
    e = senders.shape[0]
    senders = senders.astype(jnp.int32)
    receivers = receivers.astype(jnp.int32)

    blk = 1000
    grid = n // blk
    h, q, kv = pl.pallas_call(
        _proj_body,
        grid=(grid,),
        in_specs=[
            pl.BlockSpec((blk, d), lambda i: (i, 0)),
            pl.BlockSpec((d, 4 * d), lambda i: (0, 0)),
            pl.BlockSpec((1, 4 * d), lambda i: (0, 0)),
        ],
        out_specs=[pl.BlockSpec((blk, d), lambda i: (i, 0)) for _ in range(3)],
        out_shape=[jax.ShapeDtypeStruct((n, d), jnp.float32) for _ in range(3)],
    )(node_features, W_kernel, W_bias.reshape(1, 4 * d))

    zeros = jnp.zeros((n, d), jnp.float32)
    epw = e // NW
    nchunk = epw // EDGE_CHUNK
    sr = jnp.stack(
        (senders.reshape(NW, nchunk, EDGE_CHUNK),
         receivers.reshape(NW, nchunk, EDGE_CHUNK)), axis=2)
    part = _make_edge_kernel(n, e, d)(q, kv, sr, zeros)

    out = pl.pallas_call(
        _add_body,
        grid=(grid,),
        in_specs=[
            pl.BlockSpec((blk, d), lambda i: (i, 0)),
            pl.BlockSpec((1, blk, d), lambda i: (0, i, 0)),
            pl.BlockSpec((1, blk, d), lambda i: (1, i, 0)),
        ],
        out_specs=pl.BlockSpec((blk, d), lambda i: (i, 0)),
        out_shape=jax.ShapeDtypeStruct((n, d), jnp.float32),
    )(h, part, part)
    return out


# speedup vs baseline: 1.4174x; 1.0008x over previous
import functools

import jax
import jax.numpy as jnp
from jax import lax
from jax.experimental import pallas as pl
from jax.experimental.pallas import tpu as pltpu
from jax.experimental.pallas import tpu_sc as plsc

NC = 2
NS = 16
L = 16
NW = NC * NS

EDGE_CHUNK = 40


def _proj_body(x_ref, w_ref, b_ref, h_ref, q_ref, kv_ref):
    d = x_ref.shape[1]
    p = jnp.dot(x_ref[...], w_ref[...], preferred_element_type=jnp.float32)
    p = p + b_ref[...]
    h_ref[...] = p[:, 0 * d:1 * d]
    q_ref[...] = -p[:, 1 * d:2 * d]
    kb = lax.bitcast_convert_type(-p[:, 2 * d:3 * d], jnp.int32)
    vb = lax.bitcast_convert_type(p[:, 3 * d:4 * d], jnp.int32)
    kr = (kb + 0x8000) & jnp.int32(-65536)
    vr = lax.shift_right_logical(vb + 0x8000, 16)
    kv_ref[...] = lax.bitcast_convert_type(kr | vr, jnp.float32)


def _add_body(h_ref, p0_ref, p1_ref, o_ref):
    o_ref[...] = h_ref[...] + p0_ref[0] + p1_ref[0]


def _make_edge_kernel(n_nodes, n_edges, d):
    epw = n_edges // NW
    c = EDGE_CHUNK
    nchunk = epw // c
    assert nchunk % 6 == 4 and nchunk >= 16
    rpt = (n_nodes // NS) // 8 * 8
    tail = n_nodes - rpt * NS

    mesh = plsc.VectorSubcoreMesh(core_axis_name="c", subcore_axis_name="s")

    @functools.partial(
        pl.kernel,
        out_type=jax.ShapeDtypeStruct((NC, n_nodes, d), jnp.float32),
        mesh=mesh,
        compiler_params=pltpu.CompilerParams(needs_layout_passes=False),
        scratch_types=(
            [pltpu.VMEM((2, c), jnp.int32) for _ in range(6)]
            + [pltpu.VMEM((c, d), jnp.float32) for _ in range(6)]
            + [pltpu.VMEM_SHARED((n_nodes, d), jnp.float32)]
            + [pltpu.SemaphoreType.DMA for _ in range(12)]
        ),
    )
    def edge_kernel(q_hbm, kv_hbm, sr_hbm, zero_hbm, out_hbm,
                    s0, s1, s2, s3, s4, s5, qa, kva, qb, kvb, qc, kvc, acc,
                    ss0, ss1, ss2, ss3, ss4, ss5,
                    sga, sgb, sgc, sca, scb, scc):
        cid = lax.axis_index("c")
        sid = lax.axis_index("s")
        wid = sid * NC + cid
        srs = ((s0, ss0), (s1, ss1), (s2, ss2), (s3, ss3), (s4, ss4),
               (s5, ss5))
        sets = ((qa, kva, sga, sca), (qb, kvb, sgb, scb), (qc, kvc, sgc, scc))

        pltpu.sync_copy(zero_hbm.at[pl.ds(sid * rpt, rpt)],
                        acc.at[pl.ds(sid * rpt, rpt)])
        if tail:
            @pl.when(sid == NS - 1)
            def _():
                pltpu.sync_copy(zero_hbm.at[pl.ds(rpt * NS, tail)],
                                acc.at[pl.ds(rpt * NS, tail)])
        plsc.subcore_barrier()

        def fire_sr(g, slot):
            sr, sem = srs[slot]
            pltpu.async_copy(sr_hbm.at[wid, g], sr, sem)

        def wait_sr(slot):
            sr, sem = srs[slot]
            pltpu.make_async_copy(sr_hbm.at[wid, 0], sr, sem).wait()

        dummy = q_hbm.at[pl.ds(0, c)]

        def fire_gather(slot, st):
            sr, _ = srs[slot]
            qx, kvx, sem, _ = sets[st]
            pltpu.async_copy(q_hbm.at[sr.at[1]], qx, sem)
            pltpu.async_copy(kv_hbm.at[sr.at[0]], kvx, sem)

        def wait_scat(st):
            qx, kvx, _, sem = sets[st]
            pltpu.make_async_copy(dummy, kvx, sem).wait()

        def process(slot, st):
            sr, _ = srs[slot]
            qx, kvx, sem, scsem = sets[st]
            pltpu.make_async_copy(dummy, qx, sem).wait()
            pltpu.make_async_copy(dummy, kvx, sem).wait()

            def edge_body(i2, carry2):
                for u in range(4):
                    i = i2 * 4 + u
                    for j in range(d // L):
                        sl = pl.ds(j * L, L)
                        w = plsc.bitcast(kvx[i, sl], jnp.int32)
                        kf = plsc.bitcast(w, jnp.float32)
                        vf = plsc.bitcast(
                            lax.shift_left(w, jnp.full((L,), 16, jnp.int32)),
                            jnp.float32)
                        eta = 1.0 / (1.0 + jnp.exp(qx[i, sl] + kf))
                        kvx[i, sl] = eta * vf
                return carry2

            lax.fori_loop(0, c // 4, edge_body, 0)
            pltpu.async_copy(kvx, acc.at[sr.at[1]], scsem, add=True)

        for t in range(6):
            fire_sr(t, t)
        wait_sr(0)
        fire_gather(0, 0)

        for u in range(6):
            wait_sr((u + 1) % 6)
            if u >= 2:
                wait_scat((u + 1) % 3)
            fire_gather((u + 1) % 6, (u + 1) % 3)
            if u >= 2:
                fire_sr(u + 4, (u + 4) % 6)
            process(u % 6, u % 3)

        def hexad_body(i, carry):
            c0 = 6 * i
            for u in range(6):
                wait_sr((u + 1) % 6)
                wait_scat((u + 1) % 3)
                fire_gather((u + 1) % 6, (u + 1) % 3)
                fire_sr(c0 + u + 4, (u + 4) % 6)
                process(u % 6, u % 3)
            return carry

        lax.fori_loop(1, (nchunk - 10) // 6 + 1, hexad_body, 0)

        for u in range(3):
            wait_sr((u + 1) % 6)
            wait_scat((u + 1) % 3)
            fire_gather((u + 1) % 6, (u + 1) % 3)
            process(u % 6, u % 3)
        process(3, 0)
        wait_scat(1)
        wait_scat(2)
        wait_scat(0)

        plsc.subcore_barrier()
        pltpu.sync_copy(acc.at[pl.ds(sid * rpt, rpt)],
                        out_hbm.at[cid, pl.ds(sid * rpt, rpt)])
        if tail:
            @pl.when(sid == NS - 1)
            def _():
                pltpu.sync_copy(acc.at[pl.ds(rpt * NS, tail)],
                                out_hbm.at[cid, pl.ds(rpt * NS, tail)])

    return edge_kernel


def kernel(node_features, senders, receivers, W_kernel, W_bias):
    n, d = node_features.shape
    e = senders.shape[0]
    senders = senders.astype(jnp.int32)
    receivers = receivers.astype(jnp.int32)

    blk = 1000
    grid = n // blk
    h, q, kv = pl.pallas_call(
        _proj_body,
        grid=(grid,),
        in_specs=[
            pl.BlockSpec((blk, d), lambda i: (i, 0)),
            pl.BlockSpec((d, 4 * d), lambda i: (0, 0)),
            pl.BlockSpec((1, 4 * d), lambda i: (0, 0)),
        ],
        out_specs=[pl.BlockSpec((blk, d), lambda i: (i, 0)) for _ in range(3)],
        out_shape=[jax.ShapeDtypeStruct((n, d), jnp.float32) for _ in range(3)],
    )(node_features, W_kernel, W_bias.reshape(1, 4 * d))

    zeros = jnp.zeros((n, d), jnp.float32)
    epw = e // NW
    nchunk = epw // EDGE_CHUNK
    sr = jnp.stack(
        (senders.reshape(NW, nchunk, EDGE_CHUNK),
         receivers.reshape(NW, nchunk, EDGE_CHUNK)), axis=2)
    part = _make_edge_kernel(n, e, d)(q, kv, sr, zeros)

    out = pl.pallas_call(
        _add_body,
        grid=(grid,),
        in_specs=[
            pl.BlockSpec((blk, d), lambda i: (i, 0)),
            pl.BlockSpec((1, blk, d), lambda i: (0, i, 0)),
            pl.BlockSpec((1, blk, d), lambda i: (1, i, 0)),
        ],
        out_specs=pl.BlockSpec((blk, d), lambda i: (i, 0)),
        out_shape=jax.ShapeDtypeStruct((n, d), jnp.float32),
    )(h, part, part)
    return out
